# baseline (speedup 1.0000x reference)
import functools

import jax
import jax.numpy as jnp
from jax import lax
from jax.experimental import pallas as pl
from jax.experimental.pallas import tpu as pltpu
from jax.experimental.pallas import tpu_sc as plsc

N = 10000
E = 320000
H = 128
G = 128
NL = 50
ET = 4
AVG_DEG = 32.0

NC = 2
NS = 16
EPT = E // NS
BE = 80
NBLK = EPT // BE
NGRP = BE // 16


def _sc_edge_kernel(a_hbm, b4_hbm, px_hbm, py_hbm, pz_hbm,
                    row_hbm, col_hbm, et_hbm, wd_hbm, wp_hbm,
                    lanes_hbm,
                    agg_out, dp_out,
                    agg_sh, dpx_sh, dpy_sh, dpz_sh,
                    arows, brows, mrows, abuf,
                    rowv2, colv2, etv2, idxp2, idxq2, idxb2,
                    prx, pry, prz, pcx, pcy, pcz,
                    dbuf, rx2, ry2, rz2,
                    wdv, wpv, zbuf, zvec, lanesv,
                    semg, semi, sems):
    cid = lax.axis_index("c")
    sid = lax.axis_index("s")
    core0 = cid == 0
    f32 = jnp.float32

    pltpu.sync_copy(wd_hbm, wdv)
    pltpu.sync_copy(wp_hbm, wpv)
    pltpu.sync_copy(lanes_hbm, lanesv)

    zero16 = jnp.zeros((16,), f32)

    def _zrow(r, c_):
        for c in range(8):
            zbuf[r, pl.ds(16 * c, 16)] = zero16
        return c_
    lax.fori_loop(0, 32, _zrow, 0)

    def _zvec(i, c_):
        zvec[pl.ds(16 * i, 16)] = zero16
        return c_
    lax.fori_loop(0, 40, _zvec, 0)

    base_n = sid * 624
    for k in range(20):
        rows = 32 if k < 19 else 624 - 19 * 32
        pltpu.sync_copy(zbuf.at[pl.ds(0, rows)],
                        agg_sh.at[pl.ds(base_n + k * 32, rows)])

    @pl.when(sid == 0)
    def _():
        pltpu.sync_copy(zbuf.at[pl.ds(0, 16)], agg_sh.at[pl.ds(9984, 16)])

    @pl.when(core0)
    def _():
        off = sid * 624
        for ref in (dpx_sh, dpy_sh, dpz_sh):
            pltpu.sync_copy(zvec.at[pl.ds(0, 624)], ref.at[pl.ds(off, 624)])

        @pl.when(sid == 0)
        def _():
            for ref in (dpx_sh, dpy_sh, dpz_sh):
                pltpu.sync_copy(zvec.at[pl.ds(0, 16)], ref.at[pl.ds(9984, 16)])

    plsc.subcore_barrier()

    wdc = [wdv[pl.ds(16 * c, 16)] for c in range(8)]
    wpc = [wpv[pl.ds(16 * c, 16)] for c in range(8)]
    iota16 = lanesv[...]
    poff = cid * N
    ebase0 = sid * EPT

    def stage_idx(b1, p):
        base = ebase0 + b1 * BE
        pltpu.async_copy(row_hbm.at[pl.ds(base, BE)], rowv2.at[p], semi)
        pltpu.async_copy(col_hbm.at[pl.ds(base, BE)], colv2.at[p], semi)
        pltpu.async_copy(et_hbm.at[pl.ds(base, BE)], etv2.at[p], semi)

    def drain_idx(p):
        pltpu.make_async_copy(row_hbm.at[pl.ds(0, BE)], rowv2.at[p], semi).wait()
        pltpu.make_async_copy(col_hbm.at[pl.ds(0, BE)], colv2.at[p], semi).wait()
        pltpu.make_async_copy(et_hbm.at[pl.ds(0, BE)], etv2.at[p], semi).wait()

    def fire_gathers(p):
        for c in range(NGRP):
            s16 = pl.ds(16 * c, 16)
            idxp2[p, s16] = rowv2[p, s16] + poff
            idxq2[p, s16] = colv2[p, s16] + poff
            idxb2[p, s16] = colv2[p, s16] + etv2[p, s16] * N
        pltpu.async_copy(a_hbm.at[rowv2.at[p]], arows, semg)
        pltpu.async_copy(b4_hbm.at[idxb2.at[p]], brows, semg)
        pltpu.async_copy(px_hbm.at[idxp2.at[p]], prx, semg)
        pltpu.async_copy(py_hbm.at[idxp2.at[p]], pry, semg)
        pltpu.async_copy(pz_hbm.at[idxp2.at[p]], prz, semg)
        pltpu.async_copy(px_hbm.at[idxq2.at[p]], pcx, semg)
        pltpu.async_copy(py_hbm.at[idxq2.at[p]], pcy, semg)
        pltpu.async_copy(pz_hbm.at[idxq2.at[p]], pcz, semg)

    def drain_gathers(p):
        pltpu.make_async_copy(a_hbm.at[rowv2.at[p]], arows, semg).wait()
        pltpu.make_async_copy(b4_hbm.at[idxb2.at[p]], brows, semg).wait()
        pltpu.make_async_copy(px_hbm.at[idxp2.at[p]], prx, semg).wait()
        pltpu.make_async_copy(py_hbm.at[idxp2.at[p]], pry, semg).wait()
        pltpu.make_async_copy(pz_hbm.at[idxp2.at[p]], prz, semg).wait()
        pltpu.make_async_copy(px_hbm.at[idxq2.at[p]], pcx, semg).wait()
        pltpu.make_async_copy(py_hbm.at[idxq2.at[p]], pcy, semg).wait()
        pltpu.make_async_copy(pz_hbm.at[idxq2.at[p]], pcz, semg).wait()

    def fire_mscatter(p):
        pltpu.async_copy(mrows, agg_sh.at[rowv2.at[p]], sems, add=True)

    def fire_dpscatters(p):
        @pl.when(core0)
        def _():
            pltpu.async_copy(rx2.at[p], dpx_sh.at[rowv2.at[p]], sems, add=True)
            pltpu.async_copy(ry2.at[p], dpy_sh.at[rowv2.at[p]], sems, add=True)
            pltpu.async_copy(rz2.at[p], dpz_sh.at[rowv2.at[p]], sems, add=True)

    def drain_scatters(p):
        pltpu.make_async_copy(mrows, agg_sh.at[rowv2.at[p]], sems).wait()

        @pl.when(core0)
        def _():
            pltpu.make_async_copy(rx2.at[p], dpx_sh.at[rowv2.at[p]], sems).wait()
            pltpu.make_async_copy(ry2.at[p], dpy_sh.at[rowv2.at[p]], sems).wait()
            pltpu.make_async_copy(rz2.at[p], dpz_sh.at[rowv2.at[p]], sems).wait()

    stage_idx(0, 0)
    drain_idx(0)
    fire_gathers(0)

    def blk_body(b, carry):
        par = lax.rem(b, 2)
        nxt = 1 - par

        drain_gathers(par)

        for c in range(NGRP):
            s16 = pl.ds(16 * c, 16)
            dx = prx[s16] - pcx[s16]
            dy = pry[s16] - pcy[s16]
            dz = prz[s16] - pcz[s16]
            v = dx * dx + dy * dy + dz * dz + 1e-8
            iv = lax.bitcast_convert_type(v, jnp.int32)
            y = lax.bitcast_convert_type(
                jnp.int32(0x5F3759DF) - lax.shift_right_logical(iv, 1), f32)
            for _ in range(3):
                y = y * (1.5 - 0.5 * v * y * y)
            dbuf[s16] = v * y
            rx2[par, s16] = dx
            ry2[par, s16] = dy
            rz2[par, s16] = dz

        @pl.when(b > 0)
        def _():
            drain_scatters(nxt)

        @pl.when(b + 1 < NBLK)
        def _():
            stage_idx(b + 1, nxt)

        def p1_body(g, c_):
            e0 = g * 16
            d16 = dbuf[pl.ds(e0, 16)]
            for j in range(16):
                e = e0 + j
                d_s = d16[j]
                for c in range(8):
                    s16 = pl.ds(16 * c, 16)
                    mrows[e, s16] = (arows[e, s16] + brows[e, s16]
                                     + d_s * wdc[c])
            return c_
        lax.fori_loop(0, NGRP, p1_body, 0)

        @pl.when(b + 1 < NBLK)
        def _():
            drain_idx(nxt)
            fire_gathers(nxt)

        def p2_body(i, c_):
            for jj in range(2):
                e = 2 * i + jj
                acc = zero16
                for c in range(8):
                    s16 = pl.ds(16 * c, 16)
                    z = mrows[e, s16]
                    m = z / (1.0 + jnp.exp(-z))
                    mrows[e, s16] = m
                    acc = acc + m * wpc[c]
                abuf[pl.ds(e * 16, 16)] = acc
            return c_
        lax.fori_loop(0, BE // 2, p2_body, 0)

        fire_mscatter(par)

        def lanesum(vec):
            for sh in (8, 4, 2, 1):
                vec = vec + vec.at[iota16 ^ sh].get(mode="promise_in_bounds")
            return vec

        @pl.when(core0)
        def _():
            for g in range(NGRP):
                yv = zero16
                for j in range(16):
                    accv = abuf[pl.ds((g * 16 + j) * 16, 16)]
                    yv = jnp.where(iota16 == j, lanesum(accv), yv)
                t = 1.0 - 2.0 / (1.0 + jnp.exp(2.0 * yv))
                s16 = pl.ds(16 * g, 16)
                rx2[par, s16] = rx2[par, s16] * t
                ry2[par, s16] = ry2[par, s16] * t
                rz2[par, s16] = rz2[par, s16] * t

        fire_dpscatters(par)
        return carry

    lax.fori_loop(0, NBLK, blk_body, 0)
    drain_scatters((NBLK - 1) % 2)
    plsc.subcore_barrier()

    for k in range(20):
        rows = 32 if k < 19 else 624 - 19 * 32
        r0 = base_n + k * 32
        pltpu.sync_copy(agg_sh.at[pl.ds(r0, rows)], zbuf.at[pl.ds(0, rows)])
        pltpu.sync_copy(zbuf.at[pl.ds(0, rows)], agg_out.at[cid, pl.ds(r0, rows)])

    @pl.when(sid == 0)
    def _():
        pltpu.sync_copy(agg_sh.at[pl.ds(9984, 16)], zbuf.at[pl.ds(0, 16)])
        pltpu.sync_copy(zbuf.at[pl.ds(0, 16)], agg_out.at[cid, pl.ds(9984, 16)])

    @pl.when(core0)
    def _():
        off = sid * 624
        for j, ref in enumerate((dpx_sh, dpy_sh, dpz_sh)):
            pltpu.sync_copy(ref.at[pl.ds(off, 624)], zvec.at[pl.ds(0, 624)])
            pltpu.sync_copy(zvec.at[pl.ds(0, 624)],
                            dp_out.at[pl.ds(j * N + off, 624)])

        @pl.when(sid == 0)
        def _():
            for j, ref in enumerate((dpx_sh, dpy_sh, dpz_sh)):
                pltpu.sync_copy(ref.at[pl.ds(9984, 16)], zvec.at[pl.ds(0, 16)])
                pltpu.sync_copy(zvec.at[pl.ds(0, 16)],
                                dp_out.at[pl.ds(j * N + 9984, 16)])


def _sc_edge_pass(A, B4, pX, pY, pZ, row, col, et, wd, wp):
    mesh = plsc.VectorSubcoreMesh(core_axis_name="c", subcore_axis_name="s",
                                  num_cores=NC, num_subcores=NS)
    f32 = jnp.float32
    i32 = jnp.int32
    return pl.kernel(
        _sc_edge_kernel,
        out_type=[jax.ShapeDtypeStruct((NC, N, H), f32),
                  jax.ShapeDtypeStruct((3 * N,), f32)],
        mesh=mesh,
        scratch_types=[
            pltpu.VMEM_SHARED((N, H), f32),
            pltpu.VMEM_SHARED((N,), f32),
            pltpu.VMEM_SHARED((N,), f32),
            pltpu.VMEM_SHARED((N,), f32),
            pltpu.VMEM((BE, H), f32),
            pltpu.VMEM((BE, H), f32),
            pltpu.VMEM((BE, H), f32),
            pltpu.VMEM((BE * 16,), f32),
            pltpu.VMEM((2, BE), i32),
            pltpu.VMEM((2, BE), i32),
            pltpu.VMEM((2, BE), i32),
            pltpu.VMEM((2, BE), i32),
            pltpu.VMEM((2, BE), i32),
            pltpu.VMEM((2, BE), i32),
            pltpu.VMEM((BE,), f32),
            pltpu.VMEM((BE,), f32),
            pltpu.VMEM((BE,), f32),
            pltpu.VMEM((BE,), f32),
            pltpu.VMEM((BE,), f32),
            pltpu.VMEM((BE,), f32),
            pltpu.VMEM((BE,), f32),
            pltpu.VMEM((2, BE), f32),
            pltpu.VMEM((2, BE), f32),
            pltpu.VMEM((2, BE), f32),
            pltpu.VMEM((H,), f32),
            pltpu.VMEM((H,), f32),
            pltpu.VMEM((32, H), f32),
            pltpu.VMEM((640,), f32),
            pltpu.VMEM((16,), i32),
            pltpu.SemaphoreType.DMA,
            pltpu.SemaphoreType.DMA,
            pltpu.SemaphoreType.DMA,
        ],
    )(A, B4, pX, pY, pZ, row, col, et, wd, wp,
      jnp.arange(16, dtype=i32))


BN = 2000
NB = N // BN


def _k1_body(x_ref, w3_ref, a_ref, b_ref, xw_ref):
    prod = jnp.dot(x_ref[...], w3_ref[...], preferred_element_type=jnp.float32)
    a_ref[...] = prod[:, :H]
    b_ref[...] = prod[:, H:2 * H]
    xw_ref[...] = prod[:, 2 * H:]


def _tc_prep(x, W3):
    f32 = jnp.float32
    return pl.pallas_call(
        _k1_body,
        grid=(NB,),
        in_specs=[pl.BlockSpec((BN, H), lambda i: (i, 0)),
                  pl.BlockSpec((H, 3 * H), lambda i: (0, 0))],
        out_specs=[pl.BlockSpec((BN, H), lambda i: (i, 0))] * 3,
        out_shape=[jax.ShapeDtypeStruct((N, H), f32)] * 3,
    )(x, W3)


def _b4_body(b_ref, t4_ref, o_ref):
    t4row = t4_ref[pl.ds(pl.program_id(0), 1), :]
    o_ref[...] = b_ref[...][None] + t4row[None]


def _tc_b4(B, t4):
    f32 = jnp.float32
    return pl.pallas_call(
        _b4_body,
        grid=(ET, NB),
        in_specs=[pl.BlockSpec((BN, H), lambda k, j: (j, 0)),
                  pl.BlockSpec((ET, H), lambda k, j: (0, 0))],
        out_specs=pl.BlockSpec((1, BN, H), lambda k, j: (k, j, 0)),
        out_shape=jax.ShapeDtypeStruct((ET, N, H), f32),
    )(B, t4)


def _k3a_body(a1, a2, xw, dif, bat, wh2, bhr, w1, b1r, w2, b2r,
              oxg1, oxg2, old8):
    i = pl.program_id(0)
    f32 = jnp.float32
    oh = (bat[...] == lax.broadcasted_iota(jnp.int32, (BN, G), 1)).astype(f32)

    def mlp(aggb):
        xl = jax.nn.silu(xw[...] + jnp.dot(aggb, wh2[...],
                                           preferred_element_type=f32) + bhr[...])
        xl = jax.nn.silu(jnp.dot(xl, w1[...], preferred_element_type=f32) + b1r[...])
        return jnp.dot(xl, w2[...], preferred_element_type=f32) + b2r[...]

    y1 = mlp(a1[...])
    y2 = mlp(a2[...])
    d2 = dif[...] * dif[...]

    @pl.when(i == 0)
    def _():
        oxg1[...] = jnp.zeros_like(oxg1)
        oxg2[...] = jnp.zeros_like(oxg2)
        old8[...] = jnp.zeros_like(old8)

    dn = (((0,), (0,)), ((), ()))
    oxg1[...] += lax.dot_general(oh, y1, dn, preferred_element_type=f32)
    oxg2[...] += lax.dot_general(oh, y2, dn, preferred_element_type=f32)
    old8[...] += lax.dot_general(oh, d2, dn, preferred_element_type=f32)


def _tc_node(agg1, agg2, XW, dif, bat2, Wh2, bh, W1, b1, W2, b2):
    f32 = jnp.float32
    full = lambda shape: pl.BlockSpec(shape, lambda i: tuple(0 for _ in shape))
    return pl.pallas_call(
        _k3a_body,
        grid=(NB,),
        in_specs=[pl.BlockSpec((BN, H), lambda i: (i, 0)),
                  pl.BlockSpec((BN, H), lambda i: (i, 0)),
                  pl.BlockSpec((BN, H), lambda i: (i, 0)),
                  pl.BlockSpec((BN, 8), lambda i: (i, 0)),
                  pl.BlockSpec((BN, 1), lambda i: (i, 0)),
                  full((H, H)), full((1, H)), full((H, H)), full((1, H)),
                  full((H, H)), full((1, H))],
        out_specs=[full((G, H)), full((G, H)), full((G, 8))],
        out_shape=[jax.ShapeDtypeStruct((G, H), f32),
                   jax.ShapeDtypeStruct((G, H), f32),
                   jax.ShapeDtypeStruct((G, 8), f32)],
    )(agg1, agg2, XW, dif, bat2, Wh2, bh, W1, b1, W2, b2)


def _k3b_body(xg1, xg2, old8, nl, wa, wb, b1, w2, b2, o1, o2):
    f32 = jnp.float32
    h = jax.nn.silu(jnp.dot(xg2[...], wa[...], preferred_element_type=f32)
                    + jnp.dot(xg1[...], wb[...], preferred_element_type=f32)
                    + b1[...])
    ps = jnp.dot(h, w2[...], preferred_element_type=f32) + b2[...]
    mx = jnp.max(ps, axis=1, keepdims=True)
    lse = jnp.log(jnp.sum(jnp.exp(ps - mx), axis=1, keepdims=True)) + mx
    logp = ps - lse
    ohnl = nl[...] == lax.broadcasted_iota(jnp.int32, (G, NL), 1)
    val = jnp.sum(jnp.where(ohnl, logp, 0.0), axis=1)
    o1[...] = jnp.reshape(jnp.sum(old8[...]) / G, (1, 1))
    o2[...] = jnp.reshape(-jnp.mean(val), (1, 1))


def _tc_head(xg1, xg2, old8, nl2, Wa, Wb, b1, W2, b2):
    f32 = jnp.float32
    full = lambda shape: pl.BlockSpec(shape, lambda: tuple(0 for _ in shape))
    return pl.pallas_call(
        _k3b_body,
        in_specs=[full((G, H)), full((G, H)), full((G, 8)), full((G, 1)),
                  full((H, H)), full((H, H)), full((1, H)),
                  full((H, NL)), full((1, NL))],
        out_specs=[full((1, 1)), full((1, 1))],
        out_shape=[jax.ShapeDtypeStruct((1, 1), f32),
                   jax.ShapeDtypeStruct((1, 1), f32)],
    )(xg1, xg2, old8, nl2, Wa, Wb, b1, W2, b2)


def kernel(node_feature, pos, edge_index, edge_type, batch, noise_level, noise, sigmas, Wm, bm, Wh, bh, Wp, nd_W1, nd_b1, nd_W2, nd_b2, gd_W1, gd_b1, gd_W2, gd_b2, np_W1, np_b1, np_W2, np_b2):
    f32 = jnp.float32
    x = node_feature
    row, col = edge_index[0], edge_index[1]

    s = sigmas[noise_level][batch]
    ppos = pos + noise * s[:, None]

    W3 = jnp.concatenate([Wm[:H], Wm[H:2 * H], Wh[:H]], axis=1)
    t4 = Wm[2 * H + 1:] + bm[None, :]
    A, B, XW = _tc_prep(x, W3)
    B4 = _tc_b4(B, t4)

    pX = jnp.concatenate([ppos[:, 0], pos[:, 0]])
    pY = jnp.concatenate([ppos[:, 1], pos[:, 1]])
    pZ = jnp.concatenate([ppos[:, 2], pos[:, 2]])
    wd = Wm[2 * H]
    wp = Wp[:, 0]
    agg, dpT = _sc_edge_pass(A, B4.reshape(ET * N, H), pX, pY, pZ,
                             row, col, edge_type, wd, wp)

    dif3 = (dpT.reshape(3, N).T / AVG_DEG - (pos - ppos)) / s[:, None]
    dif = jnp.pad(dif3, ((0, 0), (0, 5)))
    bat2 = batch[:, None]
    xg1, xg2, old8 = _tc_node(agg[0], agg[1], XW, dif, bat2,
                              Wh[H:], bh[None, :], nd_W1, nd_b1[None, :],
                              nd_W2, nd_b2[None, :])
    o1, o2 = _tc_head(xg1, xg2, old8, noise_level[:, None],
                      np_W1[:H], np_W1[H:], np_b1[None, :], np_W2,
                      np_b2[None, :])
    return (o1.reshape(()), o2.reshape(()))


# speedup vs baseline: 1.0681x; 1.0681x over previous
import functools

import jax
import jax.numpy as jnp
from jax import lax
from jax.experimental import pallas as pl
from jax.experimental.pallas import tpu as pltpu
from jax.experimental.pallas import tpu_sc as plsc

N = 10000
E = 320000
H = 128
G = 128
NL = 50
ET = 4
AVG_DEG = 32.0

NC = 2
NS = 16
EPT = E // NS
BE = 80
NBLK = EPT // BE
NGRP = BE // 16


def _sc_edge_kernel(a_hbm, b4_hbm, px_hbm, py_hbm, pz_hbm,
                    row_hbm, col_hbm, et_hbm, wd_hbm, wp_hbm,
                    lanes_hbm,
                    agg_out, dp_out,
                    agg_sh, dpx_sh, dpy_sh, dpz_sh,
                    arows, brows, mrows, abuf,
                    rowv2, colv2, etv2, idxp2, idxq2, idxb2,
                    prx, pry, prz, pcx, pcy, pcz,
                    dbuf, rx2, ry2, rz2, scidx,
                    wdv, wpv, zbuf, zvec, lanesv,
                    semg, semi, sems, semp):
    cid = lax.axis_index("c")
    sid = lax.axis_index("s")
    core0 = cid == 0
    f32 = jnp.float32

    pltpu.sync_copy(wd_hbm, wdv)
    pltpu.sync_copy(wp_hbm, wpv)
    pltpu.sync_copy(lanes_hbm, lanesv)

    zero16 = jnp.zeros((16,), f32)

    def _zrow(r, c_):
        for c in range(8):
            zbuf[r, pl.ds(16 * c, 16)] = zero16
        return c_
    lax.fori_loop(0, 32, _zrow, 0)

    def _zvec(i, c_):
        zvec[pl.ds(16 * i, 16)] = zero16
        return c_
    lax.fori_loop(0, 40, _zvec, 0)

    base_n = sid * 624
    for k in range(20):
        rows = 32 if k < 19 else 624 - 19 * 32
        pltpu.sync_copy(zbuf.at[pl.ds(0, rows)],
                        agg_sh.at[pl.ds(base_n + k * 32, rows)])

    @pl.when(sid == 0)
    def _():
        pltpu.sync_copy(zbuf.at[pl.ds(0, 16)], agg_sh.at[pl.ds(9984, 16)])

    @pl.when(core0)
    def _():
        off = sid * 624
        for ref in (dpx_sh, dpy_sh, dpz_sh):
            pltpu.sync_copy(zvec.at[pl.ds(0, 624)], ref.at[pl.ds(off, 624)])

        @pl.when(sid == 0)
        def _():
            for ref in (dpx_sh, dpy_sh, dpz_sh):
                pltpu.sync_copy(zvec.at[pl.ds(0, 16)], ref.at[pl.ds(9984, 16)])

    plsc.subcore_barrier()

    wdc = [wdv[pl.ds(16 * c, 16)] for c in range(8)]
    iota16 = lanesv[...]
    poff = cid * N
    ebase0 = sid * EPT

    def stage_idx(b1, p):
        base = ebase0 + b1 * BE
        pltpu.async_copy(row_hbm.at[pl.ds(base, BE)], rowv2.at[p], semi)
        pltpu.async_copy(col_hbm.at[pl.ds(base, BE)], colv2.at[p], semi)
        pltpu.async_copy(et_hbm.at[pl.ds(base, BE)], etv2.at[p], semi)

    def drain_idx(p):
        pltpu.make_async_copy(row_hbm.at[pl.ds(0, BE)], rowv2.at[p], semi).wait()
        pltpu.make_async_copy(col_hbm.at[pl.ds(0, BE)], colv2.at[p], semi).wait()
        pltpu.make_async_copy(et_hbm.at[pl.ds(0, BE)], etv2.at[p], semi).wait()

    def compute_idx(p):
        for c in range(NGRP):
            s16 = pl.ds(16 * c, 16)
            idxp2[p, s16] = rowv2[p, s16] + poff
            idxq2[p, s16] = colv2[p, s16] + poff
            idxb2[p, s16] = colv2[p, s16] + etv2[p, s16] * N

    def fire_pos(p):
        pltpu.async_copy(px_hbm.at[idxp2.at[p]], prx, semp)
        pltpu.async_copy(py_hbm.at[idxp2.at[p]], pry, semp)
        pltpu.async_copy(pz_hbm.at[idxp2.at[p]], prz, semp)
        pltpu.async_copy(px_hbm.at[idxq2.at[p]], pcx, semp)
        pltpu.async_copy(py_hbm.at[idxq2.at[p]], pcy, semp)
        pltpu.async_copy(pz_hbm.at[idxq2.at[p]], pcz, semp)

    def fire_ab(p):
        pltpu.async_copy(a_hbm.at[rowv2.at[p]], arows, semg)
        pltpu.async_copy(b4_hbm.at[idxb2.at[p]], brows, semg)

    def drain_ab(p):
        pltpu.make_async_copy(a_hbm.at[rowv2.at[p]], arows, semg).wait()
        pltpu.make_async_copy(b4_hbm.at[idxb2.at[p]], brows, semg).wait()

    def drain_pos(p):
        pltpu.make_async_copy(px_hbm.at[idxp2.at[p]], prx, semp).wait()
        pltpu.make_async_copy(py_hbm.at[idxp2.at[p]], pry, semp).wait()
        pltpu.make_async_copy(pz_hbm.at[idxp2.at[p]], prz, semp).wait()
        pltpu.make_async_copy(px_hbm.at[idxq2.at[p]], pcx, semp).wait()
        pltpu.make_async_copy(py_hbm.at[idxq2.at[p]], pcy, semp).wait()
        pltpu.make_async_copy(pz_hbm.at[idxq2.at[p]], pcz, semp).wait()

    def fire_scatters(p):
        for c in range(NGRP):
            s16 = pl.ds(16 * c, 16)
            scidx[p, s16] = rowv2[p, s16]
        pltpu.async_copy(mrows, agg_sh.at[scidx.at[p]], sems, add=True)

        @pl.when(core0)
        def _():
            pltpu.async_copy(rx2.at[p], dpx_sh.at[scidx.at[p]], sems, add=True)
            pltpu.async_copy(ry2.at[p], dpy_sh.at[scidx.at[p]], sems, add=True)
            pltpu.async_copy(rz2.at[p], dpz_sh.at[scidx.at[p]], sems, add=True)

    def drain_scatters(p):
        pltpu.make_async_copy(mrows, agg_sh.at[scidx.at[p]], sems).wait()

        @pl.when(core0)
        def _():
            pltpu.make_async_copy(rx2.at[p], dpx_sh.at[scidx.at[p]], sems).wait()
            pltpu.make_async_copy(ry2.at[p], dpy_sh.at[scidx.at[p]], sems).wait()
            pltpu.make_async_copy(rz2.at[p], dpz_sh.at[scidx.at[p]], sems).wait()

    stage_idx(0, 0)
    drain_idx(0)
    compute_idx(0)
    fire_pos(0)
    fire_ab(0)

    def blk_body(b, carry):
        par = lax.rem(b, 2)
        nxt = 1 - par

        @pl.when(b + 1 < NBLK)
        def _():
            stage_idx(b + 1, nxt)

        drain_ab(par)
        drain_pos(par)

        for c in range(NGRP):
            s16 = pl.ds(16 * c, 16)
            dx = prx[s16] - pcx[s16]
            dy = pry[s16] - pcy[s16]
            dz = prz[s16] - pcz[s16]
            v = dx * dx + dy * dy + dz * dz + 1e-8
            iv = lax.bitcast_convert_type(v, jnp.int32)
            y = lax.bitcast_convert_type(
                jnp.int32(0x5F3759DF) - lax.shift_right_logical(iv, 1), f32)
            for _ in range(3):
                y = y * (1.5 - 0.5 * v * y * y)
            dbuf[s16] = v * y
            rx2[par, s16] = dx
            ry2[par, s16] = dy
            rz2[par, s16] = dz

        @pl.when(b + 1 < NBLK)
        def _():
            drain_idx(nxt)
            compute_idx(nxt)
            fire_pos(nxt)

        @pl.when(b > 0)
        def _():
            drain_scatters(nxt)

        def p1_body(g, c_):
            e0 = g * 16
            d16 = dbuf[pl.ds(e0, 16)]
            for j in range(16):
                e = e0 + j
                d_s = d16[j]
                for c in range(8):
                    s16 = pl.ds(16 * c, 16)
                    mrows[e, s16] = (arows[e, s16] + brows[e, s16]
                                     + d_s * wdc[c])
            return c_
        lax.fori_loop(0, NGRP, p1_body, 0)

        @pl.when(b + 1 < NBLK)
        def _():
            fire_ab(nxt)

        def p2_body(i, c_):
            for jj in range(2):
                e = 2 * i + jj
                acc = zero16
                for c in range(8):
                    s16 = pl.ds(16 * c, 16)
                    z = mrows[e, s16]
                    m = z / (1.0 + jnp.exp(-z))
                    mrows[e, s16] = m
                    acc = acc + m * wpv[pl.ds(16 * c, 16)]
                abuf[pl.ds(e * 16, 16)] = acc
            return c_
        lax.fori_loop(0, BE // 2, p2_body, 0)

        def lanesum(vec):
            for sh in (8, 4, 2, 1):
                vec = vec + vec.at[iota16 ^ sh].get(mode="promise_in_bounds")
            return vec

        @pl.when(core0)
        def _():
            for g in range(NGRP):
                yv = zero16
                for j in range(16):
                    accv = abuf[pl.ds((g * 16 + j) * 16, 16)]
                    yv = jnp.where(iota16 == j, lanesum(accv), yv)
                t = 1.0 - 2.0 / (1.0 + jnp.exp(2.0 * yv))
                s16 = pl.ds(16 * g, 16)
                rx2[par, s16] = rx2[par, s16] * t
                ry2[par, s16] = ry2[par, s16] * t
                rz2[par, s16] = rz2[par, s16] * t

        fire_scatters(par)
        return carry

    lax.fori_loop(0, NBLK, blk_body, 0)
    drain_scatters((NBLK - 1) % 2)
    plsc.subcore_barrier()

    for k in range(20):
        rows = 32 if k < 19 else 624 - 19 * 32
        r0 = base_n + k * 32
        pltpu.sync_copy(agg_sh.at[pl.ds(r0, rows)], zbuf.at[pl.ds(0, rows)])
        pltpu.sync_copy(zbuf.at[pl.ds(0, rows)], agg_out.at[cid, pl.ds(r0, rows)])

    @pl.when(sid == 0)
    def _():
        pltpu.sync_copy(agg_sh.at[pl.ds(9984, 16)], zbuf.at[pl.ds(0, 16)])
        pltpu.sync_copy(zbuf.at[pl.ds(0, 16)], agg_out.at[cid, pl.ds(9984, 16)])

    @pl.when(core0)
    def _():
        off = sid * 624
        for j, ref in enumerate((dpx_sh, dpy_sh, dpz_sh)):
            pltpu.sync_copy(ref.at[pl.ds(off, 624)], zvec.at[pl.ds(0, 624)])
            pltpu.sync_copy(zvec.at[pl.ds(0, 624)],
                            dp_out.at[pl.ds(j * N + off, 624)])

        @pl.when(sid == 0)
        def _():
            for j, ref in enumerate((dpx_sh, dpy_sh, dpz_sh)):
                pltpu.sync_copy(ref.at[pl.ds(9984, 16)], zvec.at[pl.ds(0, 16)])
                pltpu.sync_copy(zvec.at[pl.ds(0, 16)],
                                dp_out.at[pl.ds(j * N + 9984, 16)])


def _sc_edge_pass(A, B4, pX, pY, pZ, row, col, et, wd, wp):
    mesh = plsc.VectorSubcoreMesh(core_axis_name="c", subcore_axis_name="s",
                                  num_cores=NC, num_subcores=NS)
    f32 = jnp.float32
    i32 = jnp.int32
    return pl.kernel(
        _sc_edge_kernel,
        out_type=[jax.ShapeDtypeStruct((NC, N, H), f32),
                  jax.ShapeDtypeStruct((3 * N,), f32)],
        mesh=mesh,
        scratch_types=[
            pltpu.VMEM_SHARED((N, H), f32),
            pltpu.VMEM_SHARED((N,), f32),
            pltpu.VMEM_SHARED((N,), f32),
            pltpu.VMEM_SHARED((N,), f32),
            pltpu.VMEM((BE, H), f32),
            pltpu.VMEM((BE, H), f32),
            pltpu.VMEM((BE, H), f32),
            pltpu.VMEM((BE * 16,), f32),
            pltpu.VMEM((2, BE), i32),
            pltpu.VMEM((2, BE), i32),
            pltpu.VMEM((2, BE), i32),
            pltpu.VMEM((2, BE), i32),
            pltpu.VMEM((2, BE), i32),
            pltpu.VMEM((2, BE), i32),
            pltpu.VMEM((BE,), f32),
            pltpu.VMEM((BE,), f32),
            pltpu.VMEM((BE,), f32),
            pltpu.VMEM((BE,), f32),
            pltpu.VMEM((BE,), f32),
            pltpu.VMEM((BE,), f32),
            pltpu.VMEM((BE,), f32),
            pltpu.VMEM((2, BE), f32),
            pltpu.VMEM((2, BE), f32),
            pltpu.VMEM((2, BE), f32),
            pltpu.VMEM((2, BE), i32),
            pltpu.VMEM((H,), f32),
            pltpu.VMEM((H,), f32),
            pltpu.VMEM((32, H), f32),
            pltpu.VMEM((640,), f32),
            pltpu.VMEM((16,), i32),
            pltpu.SemaphoreType.DMA,
            pltpu.SemaphoreType.DMA,
            pltpu.SemaphoreType.DMA,
            pltpu.SemaphoreType.DMA,
        ],
    )(A, B4, pX, pY, pZ, row, col, et, wd, wp,
      jnp.arange(16, dtype=i32))


BN = 2000
NB = N // BN


def _k1_body(x_ref, w3_ref, a_ref, b_ref, xw_ref):
    prod = jnp.dot(x_ref[...], w3_ref[...], preferred_element_type=jnp.float32)
    a_ref[...] = prod[:, :H]
    b_ref[...] = prod[:, H:2 * H]
    xw_ref[...] = prod[:, 2 * H:]


def _tc_prep(x, W3):
    f32 = jnp.float32
    return pl.pallas_call(
        _k1_body,
        grid=(NB,),
        in_specs=[pl.BlockSpec((BN, H), lambda i: (i, 0)),
                  pl.BlockSpec((H, 3 * H), lambda i: (0, 0))],
        out_specs=[pl.BlockSpec((BN, H), lambda i: (i, 0))] * 3,
        out_shape=[jax.ShapeDtypeStruct((N, H), f32)] * 3,
    )(x, W3)


def _b4_body(b_ref, t4_ref, o_ref):
    t4row = t4_ref[pl.ds(pl.program_id(0), 1), :]
    o_ref[...] = b_ref[...][None] + t4row[None]


def _tc_b4(B, t4):
    f32 = jnp.float32
    return pl.pallas_call(
        _b4_body,
        grid=(ET, NB),
        in_specs=[pl.BlockSpec((BN, H), lambda k, j: (j, 0)),
                  pl.BlockSpec((ET, H), lambda k, j: (0, 0))],
        out_specs=pl.BlockSpec((1, BN, H), lambda k, j: (k, j, 0)),
        out_shape=jax.ShapeDtypeStruct((ET, N, H), f32),
    )(B, t4)


def _k3a_body(a1, a2, xw, dif, bat, wh2, bhr, w1, b1r, w2, b2r,
              oxg1, oxg2, old8):
    i = pl.program_id(0)
    f32 = jnp.float32
    oh = (bat[...] == lax.broadcasted_iota(jnp.int32, (BN, G), 1)).astype(f32)

    def mlp(aggb):
        xl = jax.nn.silu(xw[...] + jnp.dot(aggb, wh2[...],
                                           preferred_element_type=f32) + bhr[...])
        xl = jax.nn.silu(jnp.dot(xl, w1[...], preferred_element_type=f32) + b1r[...])
        return jnp.dot(xl, w2[...], preferred_element_type=f32) + b2r[...]

    y1 = mlp(a1[...])
    y2 = mlp(a2[...])
    d2 = dif[...] * dif[...]

    @pl.when(i == 0)
    def _():
        oxg1[...] = jnp.zeros_like(oxg1)
        oxg2[...] = jnp.zeros_like(oxg2)
        old8[...] = jnp.zeros_like(old8)

    dn = (((0,), (0,)), ((), ()))
    oxg1[...] += lax.dot_general(oh, y1, dn, preferred_element_type=f32)
    oxg2[...] += lax.dot_general(oh, y2, dn, preferred_element_type=f32)
    old8[...] += lax.dot_general(oh, d2, dn, preferred_element_type=f32)


def _tc_node(agg1, agg2, XW, dif, bat2, Wh2, bh, W1, b1, W2, b2):
    f32 = jnp.float32
    full = lambda shape: pl.BlockSpec(shape, lambda i: tuple(0 for _ in shape))
    return pl.pallas_call(
        _k3a_body,
        grid=(NB,),
        in_specs=[pl.BlockSpec((BN, H), lambda i: (i, 0)),
                  pl.BlockSpec((BN, H), lambda i: (i, 0)),
                  pl.BlockSpec((BN, H), lambda i: (i, 0)),
                  pl.BlockSpec((BN, 8), lambda i: (i, 0)),
                  pl.BlockSpec((BN, 1), lambda i: (i, 0)),
                  full((H, H)), full((1, H)), full((H, H)), full((1, H)),
                  full((H, H)), full((1, H))],
        out_specs=[full((G, H)), full((G, H)), full((G, 8))],
        out_shape=[jax.ShapeDtypeStruct((G, H), f32),
                   jax.ShapeDtypeStruct((G, H), f32),
                   jax.ShapeDtypeStruct((G, 8), f32)],
    )(agg1, agg2, XW, dif, bat2, Wh2, bh, W1, b1, W2, b2)


def _k3b_body(xg1, xg2, old8, nl, wa, wb, b1, w2, b2, o1, o2):
    f32 = jnp.float32
    h = jax.nn.silu(jnp.dot(xg2[...], wa[...], preferred_element_type=f32)
                    + jnp.dot(xg1[...], wb[...], preferred_element_type=f32)
                    + b1[...])
    ps = jnp.dot(h, w2[...], preferred_element_type=f32) + b2[...]
    mx = jnp.max(ps, axis=1, keepdims=True)
    lse = jnp.log(jnp.sum(jnp.exp(ps - mx), axis=1, keepdims=True)) + mx
    logp = ps - lse
    ohnl = nl[...] == lax.broadcasted_iota(jnp.int32, (G, NL), 1)
    val = jnp.sum(jnp.where(ohnl, logp, 0.0), axis=1)
    o1[...] = jnp.reshape(jnp.sum(old8[...]) / G, (1, 1))
    o2[...] = jnp.reshape(-jnp.mean(val), (1, 1))


def _tc_head(xg1, xg2, old8, nl2, Wa, Wb, b1, W2, b2):
    f32 = jnp.float32
    full = lambda shape: pl.BlockSpec(shape, lambda: tuple(0 for _ in shape))
    return pl.pallas_call(
        _k3b_body,
        in_specs=[full((G, H)), full((G, H)), full((G, 8)), full((G, 1)),
                  full((H, H)), full((H, H)), full((1, H)),
                  full((H, NL)), full((1, NL))],
        out_specs=[full((1, 1)), full((1, 1))],
        out_shape=[jax.ShapeDtypeStruct((1, 1), f32),
                   jax.ShapeDtypeStruct((1, 1), f32)],
    )(xg1, xg2, old8, nl2, Wa, Wb, b1, W2, b2)


def kernel(node_feature, pos, edge_index, edge_type, batch, noise_level, noise, sigmas, Wm, bm, Wh, bh, Wp, nd_W1, nd_b1, nd_W2, nd_b2, gd_W1, gd_b1, gd_W2, gd_b2, np_W1, np_b1, np_W2, np_b2):
    f32 = jnp.float32
    x = node_feature
    row, col = edge_index[0], edge_index[1]

    s = sigmas[noise_level][batch]
    ppos = pos + noise * s[:, None]

    W3 = jnp.concatenate([Wm[:H], Wm[H:2 * H], Wh[:H]], axis=1)
    t4 = Wm[2 * H + 1:] + bm[None, :]
    A, B, XW = _tc_prep(x, W3)
    B4 = _tc_b4(B, t4)

    pX = jnp.concatenate([ppos[:, 0], pos[:, 0]])
    pY = jnp.concatenate([ppos[:, 1], pos[:, 1]])
    pZ = jnp.concatenate([ppos[:, 2], pos[:, 2]])
    wd = Wm[2 * H]
    wp = Wp[:, 0]
    agg, dpT = _sc_edge_pass(A, B4.reshape(ET * N, H), pX, pY, pZ,
                             row, col, edge_type, wd, wp)

    dif3 = (dpT.reshape(3, N).T / AVG_DEG - (pos - ppos)) / s[:, None]
    dif = jnp.pad(dif3, ((0, 0), (0, 5)))
    bat2 = batch[:, None]
    xg1, xg2, old8 = _tc_node(agg[0], agg[1], XW, dif, bat2,
                              Wh[H:], bh[None, :], nd_W1, nd_b1[None, :],
                              nd_W2, nd_b2[None, :])
    o1, o2 = _tc_head(xg1, xg2, old8, noise_level[:, None],
                      np_W1[:H], np_W1[H:], np_b1[None, :], np_W2,
                      np_b2[None, :])
    return (o1.reshape(()), o2.reshape(()))


# speedup vs baseline: 1.1100x; 1.0392x over previous
import functools

import jax
import jax.numpy as jnp
from jax import lax
from jax.experimental import pallas as pl
from jax.experimental.pallas import tpu as pltpu
from jax.experimental.pallas import tpu_sc as plsc

N = 10000
E = 320000
H = 128
G = 128
NL = 50
ET = 4
AVG_DEG = 32.0

NC = 2
NS = 16
EPT = E // NS
BE = 80
NBLK = EPT // BE
NGRP = BE // 16


def _sc_edge_kernel(a_hbm, b4_hbm, px_hbm, py_hbm, pz_hbm,
                    row_hbm, col_hbm, et_hbm, wd_hbm, wp_hbm,
                    lanes_hbm,
                    agg_out, dp_out,
                    agg_sh, dpx_sh, dpy_sh, dpz_sh,
                    arows, brows, mrows, abuf,
                    rowv2, colv2, etv2, idxp2, idxq2, idxb2,
                    prx, pry, prz, pcx, pcy, pcz,
                    dbuf, rx2, ry2, rz2,
                    wdv, wpv, zbuf, zvec, lanesv,
                    semg, semi, sems):
    cid = lax.axis_index("c")
    sid = lax.axis_index("s")
    core0 = cid == 0
    f32 = jnp.float32

    pltpu.sync_copy(wd_hbm, wdv)
    pltpu.sync_copy(wp_hbm, wpv)
    pltpu.sync_copy(lanes_hbm, lanesv)

    zero16 = jnp.zeros((16,), f32)

    def _zrow(r, c_):
        for c in range(8):
            zbuf[r, pl.ds(16 * c, 16)] = zero16
        return c_
    lax.fori_loop(0, 32, _zrow, 0)

    def _zvec(i, c_):
        zvec[pl.ds(16 * i, 16)] = zero16
        return c_
    lax.fori_loop(0, 40, _zvec, 0)

    base_n = sid * 624
    for k in range(20):
        rows = 32 if k < 19 else 624 - 19 * 32
        pltpu.sync_copy(zbuf.at[pl.ds(0, rows)],
                        agg_sh.at[pl.ds(base_n + k * 32, rows)])

    @pl.when(sid == 0)
    def _():
        pltpu.sync_copy(zbuf.at[pl.ds(0, 16)], agg_sh.at[pl.ds(9984, 16)])

    @pl.when(core0)
    def _():
        off = sid * 624
        for ref in (dpx_sh, dpy_sh, dpz_sh):
            pltpu.sync_copy(zvec.at[pl.ds(0, 624)], ref.at[pl.ds(off, 624)])

        @pl.when(sid == 0)
        def _():
            for ref in (dpx_sh, dpy_sh, dpz_sh):
                pltpu.sync_copy(zvec.at[pl.ds(0, 16)], ref.at[pl.ds(9984, 16)])

    plsc.subcore_barrier()

    wdc = [wdv[pl.ds(16 * c, 16)] for c in range(8)]
    iota16 = lanesv[...]
    poff = cid * N
    ebase0 = sid * EPT

    def stage_idx(b1, p):
        base = ebase0 + b1 * BE
        pltpu.async_copy(row_hbm.at[pl.ds(base, BE)], rowv2.at[p], semi)
        pltpu.async_copy(col_hbm.at[pl.ds(base, BE)], colv2.at[p], semi)
        pltpu.async_copy(et_hbm.at[pl.ds(base, BE)], etv2.at[p], semi)

    def drain_idx(p):
        pltpu.make_async_copy(row_hbm.at[pl.ds(0, BE)], rowv2.at[p], semi).wait()
        pltpu.make_async_copy(col_hbm.at[pl.ds(0, BE)], colv2.at[p], semi).wait()
        pltpu.make_async_copy(et_hbm.at[pl.ds(0, BE)], etv2.at[p], semi).wait()

    def fire_gathers(p):
        for c in range(NGRP):
            s16 = pl.ds(16 * c, 16)
            idxp2[p, s16] = rowv2[p, s16] + poff
            idxq2[p, s16] = colv2[p, s16] + poff
            idxb2[p, s16] = colv2[p, s16] + etv2[p, s16] * N
        pltpu.async_copy(a_hbm.at[rowv2.at[p]], arows, semg)
        pltpu.async_copy(b4_hbm.at[idxb2.at[p]], brows, semg)
        pltpu.async_copy(px_hbm.at[idxp2.at[p]], prx, semg)
        pltpu.async_copy(py_hbm.at[idxp2.at[p]], pry, semg)
        pltpu.async_copy(pz_hbm.at[idxp2.at[p]], prz, semg)
        pltpu.async_copy(px_hbm.at[idxq2.at[p]], pcx, semg)
        pltpu.async_copy(py_hbm.at[idxq2.at[p]], pcy, semg)
        pltpu.async_copy(pz_hbm.at[idxq2.at[p]], pcz, semg)

    def drain_gathers(p):
        pltpu.make_async_copy(a_hbm.at[rowv2.at[p]], arows, semg).wait()
        pltpu.make_async_copy(b4_hbm.at[idxb2.at[p]], brows, semg).wait()
        pltpu.make_async_copy(px_hbm.at[idxp2.at[p]], prx, semg).wait()
        pltpu.make_async_copy(py_hbm.at[idxp2.at[p]], pry, semg).wait()
        pltpu.make_async_copy(pz_hbm.at[idxp2.at[p]], prz, semg).wait()
        pltpu.make_async_copy(px_hbm.at[idxq2.at[p]], pcx, semg).wait()
        pltpu.make_async_copy(py_hbm.at[idxq2.at[p]], pcy, semg).wait()
        pltpu.make_async_copy(pz_hbm.at[idxq2.at[p]], pcz, semg).wait()

    def fire_scatters(p):
        pltpu.async_copy(mrows, agg_sh.at[rowv2.at[p]], sems, add=True)

        @pl.when(core0)
        def _():
            pltpu.async_copy(rx2.at[p], dpx_sh.at[rowv2.at[p]], sems, add=True)
            pltpu.async_copy(ry2.at[p], dpy_sh.at[rowv2.at[p]], sems, add=True)
            pltpu.async_copy(rz2.at[p], dpz_sh.at[rowv2.at[p]], sems, add=True)

    def drain_scatters(p):
        pltpu.make_async_copy(mrows, agg_sh.at[rowv2.at[p]], sems).wait()

        @pl.when(core0)
        def _():
            pltpu.make_async_copy(rx2.at[p], dpx_sh.at[rowv2.at[p]], sems).wait()
            pltpu.make_async_copy(ry2.at[p], dpy_sh.at[rowv2.at[p]], sems).wait()
            pltpu.make_async_copy(rz2.at[p], dpz_sh.at[rowv2.at[p]], sems).wait()

    stage_idx(0, 0)
    drain_idx(0)
    fire_gathers(0)

    def blk_body(b, carry):
        par = lax.rem(b, 2)
        nxt = 1 - par

        drain_gathers(par)

        for c in range(NGRP):
            s16 = pl.ds(16 * c, 16)
            dx = prx[s16] - pcx[s16]
            dy = pry[s16] - pcy[s16]
            dz = prz[s16] - pcz[s16]
            v = dx * dx + dy * dy + dz * dz + 1e-8
            iv = lax.bitcast_convert_type(v, jnp.int32)
            y = lax.bitcast_convert_type(
                jnp.int32(0x5F3759DF) - lax.shift_right_logical(iv, 1), f32)
            for _ in range(3):
                y = y * (1.5 - 0.5 * v * y * y)
            dbuf[s16] = v * y
            rx2[par, s16] = dx
            ry2[par, s16] = dy
            rz2[par, s16] = dz

        @pl.when(b > 0)
        def _():
            drain_scatters(nxt)

        @pl.when(b + 1 < NBLK)
        def _():
            stage_idx(b + 1, nxt)

        def p1_body(g, c_):
            e0 = g * 16
            d16 = dbuf[pl.ds(e0, 16)]
            for j in range(16):
                e = e0 + j
                d_s = d16[j]
                for c in range(8):
                    s16 = pl.ds(16 * c, 16)
                    mrows[e, s16] = (arows[e, s16] + brows[e, s16]
                                     + d_s * wdc[c])
            return c_
        lax.fori_loop(0, NGRP, p1_body, 0)

        @pl.when(b + 1 < NBLK)
        def _():
            drain_idx(nxt)
            fire_gathers(nxt)

        def p2_body(i, c_):
            for jj in range(2):
                e = 2 * i + jj
                acc = zero16
                for c in range(8):
                    s16 = pl.ds(16 * c, 16)
                    z = mrows[e, s16]
                    m = z / (1.0 + jnp.exp(-z))
                    mrows[e, s16] = m
                    acc = acc + m * wpv[pl.ds(16 * c, 16)]
                abuf[pl.ds(e * 16, 16)] = acc
            return c_
        lax.fori_loop(0, BE // 2, p2_body, 0)

        def lanesum(vec):
            for sh in (8, 4, 2, 1):
                vec = vec + vec.at[iota16 ^ sh].get(mode="promise_in_bounds")
            return vec

        @pl.when(core0)
        def _():
            for g in range(NGRP):
                yv = zero16
                for j in range(16):
                    accv = abuf[pl.ds((g * 16 + j) * 16, 16)]
                    yv = jnp.where(iota16 == j, lanesum(accv), yv)
                t = 1.0 - 2.0 / (1.0 + jnp.exp(2.0 * yv))
                s16 = pl.ds(16 * g, 16)
                rx2[par, s16] = rx2[par, s16] * t
                ry2[par, s16] = ry2[par, s16] * t
                rz2[par, s16] = rz2[par, s16] * t

        fire_scatters(par)
        return carry

    lax.fori_loop(0, NBLK, blk_body, 0)
    drain_scatters((NBLK - 1) % 2)
    plsc.subcore_barrier()

    for k in range(20):
        rows = 32 if k < 19 else 624 - 19 * 32
        r0 = base_n + k * 32
        pltpu.sync_copy(agg_sh.at[pl.ds(r0, rows)], zbuf.at[pl.ds(0, rows)])
        pltpu.sync_copy(zbuf.at[pl.ds(0, rows)], agg_out.at[cid, pl.ds(r0, rows)])

    @pl.when(sid == 0)
    def _():
        pltpu.sync_copy(agg_sh.at[pl.ds(9984, 16)], zbuf.at[pl.ds(0, 16)])
        pltpu.sync_copy(zbuf.at[pl.ds(0, 16)], agg_out.at[cid, pl.ds(9984, 16)])

    @pl.when(core0)
    def _():
        off = sid * 624
        for j, ref in enumerate((dpx_sh, dpy_sh, dpz_sh)):
            pltpu.sync_copy(ref.at[pl.ds(off, 624)], zvec.at[pl.ds(0, 624)])
            pltpu.sync_copy(zvec.at[pl.ds(0, 624)],
                            dp_out.at[pl.ds(j * N + off, 624)])

        @pl.when(sid == 0)
        def _():
            for j, ref in enumerate((dpx_sh, dpy_sh, dpz_sh)):
                pltpu.sync_copy(ref.at[pl.ds(9984, 16)], zvec.at[pl.ds(0, 16)])
                pltpu.sync_copy(zvec.at[pl.ds(0, 16)],
                                dp_out.at[pl.ds(j * N + 9984, 16)])


def _sc_edge_pass(A, B4, pX, pY, pZ, row, col, et, wd, wp):
    mesh = plsc.VectorSubcoreMesh(core_axis_name="c", subcore_axis_name="s",
                                  num_cores=NC, num_subcores=NS)
    f32 = jnp.float32
    i32 = jnp.int32
    return pl.kernel(
        _sc_edge_kernel,
        out_type=[jax.ShapeDtypeStruct((NC, N, H), f32),
                  jax.ShapeDtypeStruct((3 * N,), f32)],
        mesh=mesh,
        scratch_types=[
            pltpu.VMEM_SHARED((N, H), f32),
            pltpu.VMEM_SHARED((N,), f32),
            pltpu.VMEM_SHARED((N,), f32),
            pltpu.VMEM_SHARED((N,), f32),
            pltpu.VMEM((BE, H), f32),
            pltpu.VMEM((BE, H), f32),
            pltpu.VMEM((BE, H), f32),
            pltpu.VMEM((BE * 16,), f32),
            pltpu.VMEM((2, BE), i32),
            pltpu.VMEM((2, BE), i32),
            pltpu.VMEM((2, BE), i32),
            pltpu.VMEM((2, BE), i32),
            pltpu.VMEM((2, BE), i32),
            pltpu.VMEM((2, BE), i32),
            pltpu.VMEM((BE,), f32),
            pltpu.VMEM((BE,), f32),
            pltpu.VMEM((BE,), f32),
            pltpu.VMEM((BE,), f32),
            pltpu.VMEM((BE,), f32),
            pltpu.VMEM((BE,), f32),
            pltpu.VMEM((BE,), f32),
            pltpu.VMEM((2, BE), f32),
            pltpu.VMEM((2, BE), f32),
            pltpu.VMEM((2, BE), f32),
            pltpu.VMEM((H,), f32),
            pltpu.VMEM((H,), f32),
            pltpu.VMEM((32, H), f32),
            pltpu.VMEM((640,), f32),
            pltpu.VMEM((16,), i32),
            pltpu.SemaphoreType.DMA,
            pltpu.SemaphoreType.DMA,
            pltpu.SemaphoreType.DMA,
        ],
    )(A, B4, pX, pY, pZ, row, col, et, wd, wp,
      jnp.arange(16, dtype=i32))


BN = 2000
NB = N // BN


def _k1_body(x_ref, w3_ref, a_ref, b_ref, xw_ref):
    prod = jnp.dot(x_ref[...], w3_ref[...], preferred_element_type=jnp.float32)
    a_ref[...] = prod[:, :H]
    b_ref[...] = prod[:, H:2 * H]
    xw_ref[...] = prod[:, 2 * H:]


def _tc_prep(x, W3):
    f32 = jnp.float32
    return pl.pallas_call(
        _k1_body,
        grid=(NB,),
        in_specs=[pl.BlockSpec((BN, H), lambda i: (i, 0)),
                  pl.BlockSpec((H, 3 * H), lambda i: (0, 0))],
        out_specs=[pl.BlockSpec((BN, H), lambda i: (i, 0))] * 3,
        out_shape=[jax.ShapeDtypeStruct((N, H), f32)] * 3,
    )(x, W3)


def _b4_body(b_ref, t4_ref, o_ref):
    t4row = t4_ref[pl.ds(pl.program_id(0), 1), :]
    o_ref[...] = b_ref[...][None] + t4row[None]


def _tc_b4(B, t4):
    f32 = jnp.float32
    return pl.pallas_call(
        _b4_body,
        grid=(ET, NB),
        in_specs=[pl.BlockSpec((BN, H), lambda k, j: (j, 0)),
                  pl.BlockSpec((ET, H), lambda k, j: (0, 0))],
        out_specs=pl.BlockSpec((1, BN, H), lambda k, j: (k, j, 0)),
        out_shape=jax.ShapeDtypeStruct((ET, N, H), f32),
    )(B, t4)


def _k3a_body(a1, a2, xw, dif, bat, wh2, bhr, w1, b1r, w2, b2r,
              oxg1, oxg2, old8):
    i = pl.program_id(0)
    f32 = jnp.float32
    oh = (bat[...] == lax.broadcasted_iota(jnp.int32, (BN, G), 1)).astype(f32)

    def mlp(aggb):
        xl = jax.nn.silu(xw[...] + jnp.dot(aggb, wh2[...],
                                           preferred_element_type=f32) + bhr[...])
        xl = jax.nn.silu(jnp.dot(xl, w1[...], preferred_element_type=f32) + b1r[...])
        return jnp.dot(xl, w2[...], preferred_element_type=f32) + b2r[...]

    y1 = mlp(a1[...])
    y2 = mlp(a2[...])
    d2 = dif[...] * dif[...]

    @pl.when(i == 0)
    def _():
        oxg1[...] = jnp.zeros_like(oxg1)
        oxg2[...] = jnp.zeros_like(oxg2)
        old8[...] = jnp.zeros_like(old8)

    dn = (((0,), (0,)), ((), ()))
    oxg1[...] += lax.dot_general(oh, y1, dn, preferred_element_type=f32)
    oxg2[...] += lax.dot_general(oh, y2, dn, preferred_element_type=f32)
    old8[...] += lax.dot_general(oh, d2, dn, preferred_element_type=f32)


def _tc_node(agg1, agg2, XW, dif, bat2, Wh2, bh, W1, b1, W2, b2):
    f32 = jnp.float32
    full = lambda shape: pl.BlockSpec(shape, lambda i: tuple(0 for _ in shape))
    return pl.pallas_call(
        _k3a_body,
        grid=(NB,),
        in_specs=[pl.BlockSpec((BN, H), lambda i: (i, 0)),
                  pl.BlockSpec((BN, H), lambda i: (i, 0)),
                  pl.BlockSpec((BN, H), lambda i: (i, 0)),
                  pl.BlockSpec((BN, 8), lambda i: (i, 0)),
                  pl.BlockSpec((BN, 1), lambda i: (i, 0)),
                  full((H, H)), full((1, H)), full((H, H)), full((1, H)),
                  full((H, H)), full((1, H))],
        out_specs=[full((G, H)), full((G, H)), full((G, 8))],
        out_shape=[jax.ShapeDtypeStruct((G, H), f32),
                   jax.ShapeDtypeStruct((G, H), f32),
                   jax.ShapeDtypeStruct((G, 8), f32)],
    )(agg1, agg2, XW, dif, bat2, Wh2, bh, W1, b1, W2, b2)


def _k3b_body(xg1, xg2, old8, nl, wa, wb, b1, w2, b2, o1, o2):
    f32 = jnp.float32
    h = jax.nn.silu(jnp.dot(xg2[...], wa[...], preferred_element_type=f32)
                    + jnp.dot(xg1[...], wb[...], preferred_element_type=f32)
                    + b1[...])
    ps = jnp.dot(h, w2[...], preferred_element_type=f32) + b2[...]
    mx = jnp.max(ps, axis=1, keepdims=True)
    lse = jnp.log(jnp.sum(jnp.exp(ps - mx), axis=1, keepdims=True)) + mx
    logp = ps - lse
    ohnl = nl[...] == lax.broadcasted_iota(jnp.int32, (G, NL), 1)
    val = jnp.sum(jnp.where(ohnl, logp, 0.0), axis=1)
    o1[...] = jnp.reshape(jnp.sum(old8[...]) / G, (1, 1))
    o2[...] = jnp.reshape(-jnp.mean(val), (1, 1))


def _tc_head(xg1, xg2, old8, nl2, Wa, Wb, b1, W2, b2):
    f32 = jnp.float32
    full = lambda shape: pl.BlockSpec(shape, lambda: tuple(0 for _ in shape))
    return pl.pallas_call(
        _k3b_body,
        in_specs=[full((G, H)), full((G, H)), full((G, 8)), full((G, 1)),
                  full((H, H)), full((H, H)), full((1, H)),
                  full((H, NL)), full((1, NL))],
        out_specs=[full((1, 1)), full((1, 1))],
        out_shape=[jax.ShapeDtypeStruct((1, 1), f32),
                   jax.ShapeDtypeStruct((1, 1), f32)],
    )(xg1, xg2, old8, nl2, Wa, Wb, b1, W2, b2)


def kernel(node_feature, pos, edge_index, edge_type, batch, noise_level, noise, sigmas, Wm, bm, Wh, bh, Wp, nd_W1, nd_b1, nd_W2, nd_b2, gd_W1, gd_b1, gd_W2, gd_b2, np_W1, np_b1, np_W2, np_b2):
    f32 = jnp.float32
    x = node_feature
    row, col = edge_index[0], edge_index[1]

    s = sigmas[noise_level][batch]
    ppos = pos + noise * s[:, None]

    W3 = jnp.concatenate([Wm[:H], Wm[H:2 * H], Wh[:H]], axis=1)
    t4 = Wm[2 * H + 1:] + bm[None, :]
    A, B, XW = _tc_prep(x, W3)
    B4 = _tc_b4(B, t4)

    pX = jnp.concatenate([ppos[:, 0], pos[:, 0]])
    pY = jnp.concatenate([ppos[:, 1], pos[:, 1]])
    pZ = jnp.concatenate([ppos[:, 2], pos[:, 2]])
    wd = Wm[2 * H]
    wp = Wp[:, 0]
    agg, dpT = _sc_edge_pass(A, B4.reshape(ET * N, H), pX, pY, pZ,
                             row, col, edge_type, wd, wp)

    dif3 = (dpT.reshape(3, N).T / AVG_DEG - (pos - ppos)) / s[:, None]
    dif = jnp.pad(dif3, ((0, 0), (0, 5)))
    bat2 = batch[:, None]
    xg1, xg2, old8 = _tc_node(agg[0], agg[1], XW, dif, bat2,
                              Wh[H:], bh[None, :], nd_W1, nd_b1[None, :],
                              nd_W2, nd_b2[None, :])
    o1, o2 = _tc_head(xg1, xg2, old8, noise_level[:, None],
                      np_W1[:H], np_W1[H:], np_b1[None, :], np_W2,
                      np_b2[None, :])
    return (o1.reshape(()), o2.reshape(()))
